# baseline (device time: 18591 ns/iter reference)
import jax
import jax.numpy as jnp
from jax import lax
from jax.experimental import pallas as pl
from jax.experimental.pallas import tpu as pltpu

N_DEV = 8
B = 2
SQ = 128
SKV = 128
HQ = 32
DH = 64
DM = 512
H_LOC = HQ // N_DEV
CH = H_LOC * DH
RB = (B * SQ) // N_DEV
BLK_PER_B = SQ // RB

_XOR_ORDER = [6, 2, 5, 7, 1, 3, 4]


def kernel(x, Wq, K_ext, V_ext, Wo):
    my = lax.axis_index("i")
    z = jnp.int32(0)
    bf = jnp.bfloat16
    wq_loc = lax.dynamic_slice(Wq, (z, my * CH), (DM, CH)).astype(bf)
    wo_loc = lax.dynamic_slice(Wo, (my * CH, z), (CH, DM)).astype(bf)
    xm = x.reshape(B * SQ, DM)
    kb = K_ext.astype(bf)
    vb = V_ext.astype(bf)

    def body(x_ref, wq_ref, k_ref, v_ref, wo_ref, out_ref,
             sbuf, rs_buf, ag_buf,
             rs_send, rs_recv, ag_send, ag_recv):
        me = lax.axis_index("i")

        barrier = pltpu.get_barrier_semaphore()
        for d in range(1, N_DEV):
            pl.semaphore_signal(
                barrier, inc=1,
                device_id=(me ^ d,),
                device_id_type=pl.DeviceIdType.MESH,
            )

        xb = x_ref[:].astype(jnp.bfloat16)
        q = jnp.dot(xb, wq_ref[:],
                    preferred_element_type=jnp.float32)
        q = (q * 0.125).astype(jnp.bfloat16)

        ii = lax.broadcasted_iota(jnp.int32, (SQ, SKV), 0)
        jj = lax.broadcasted_iota(jnp.int32, (SQ, SKV), 1)
        mask = (jj // 64) <= (ii // 64)

        kv = k_ref[:]
        vv = v_ref[:]
        wo = wo_ref[:]

        for b in range(B):
            blocks = []
            for h in range(H_LOC):
                qbh = q[b * SQ:(b + 1) * SQ, h * DH:(h + 1) * DH]
                kbh = kv[b, :, h, :]
                vbh = vv[b, :, h, :]
                scores = lax.dot_general(
                    qbh, kbh, (((1,), (1,)), ((), ())),
                    preferred_element_type=jnp.float32,
                )
                w = jnp.exp(jnp.where(mask, scores, -1e9))
                s = jnp.sum(w, axis=1, keepdims=True)
                ctx = jnp.dot(w.astype(jnp.bfloat16), vbh,
                              preferred_element_type=jnp.float32)
                blocks.append((ctx / s).astype(jnp.bfloat16))
            ctx_b = jnp.concatenate(blocks, axis=1)
            partial_b = jnp.dot(ctx_b, wo,
                                preferred_element_type=jnp.float32)
            sbuf[b * BLK_PER_B:(b + 1) * BLK_PER_B] = (
                partial_b.reshape(BLK_PER_B, RB, DM).astype(jnp.bfloat16))

            if b == 0:
                pl.semaphore_wait(barrier, N_DEV - 1)
            for d in _XOR_ORDER:
                p = me ^ d

                @pl.when((p // BLK_PER_B) == b)
                def _():
                    pltpu.make_async_remote_copy(
                        src_ref=sbuf.at[p],
                        dst_ref=rs_buf.at[me],
                        send_sem=rs_send.at[p],
                        recv_sem=rs_recv.at[me],
                        device_id=(p,),
                        device_id_type=pl.DeviceIdType.MESH,
                    ).start()

        rs_buf[me] = sbuf[me]

        for d in _XOR_ORDER:
            p = me ^ d
            pltpu.make_async_remote_copy(
                src_ref=sbuf.at[p],
                dst_ref=rs_buf.at[p],
                send_sem=rs_send.at[p],
                recv_sem=rs_recv.at[p],
                device_id=(p,),
                device_id_type=pl.DeviceIdType.MESH,
            ).wait_recv()

        red = jnp.sum(rs_buf[:].astype(jnp.float32), axis=0)
        ag_buf[me] = red.astype(jnp.bfloat16)

        ag_descs = []
        for d in _XOR_ORDER:
            p = me ^ d
            rdma = pltpu.make_async_remote_copy(
                src_ref=ag_buf.at[me],
                dst_ref=ag_buf.at[me],
                send_sem=ag_send.at[p],
                recv_sem=ag_recv.at[me],
                device_id=(p,),
                device_id_type=pl.DeviceIdType.MESH,
            )
            rdma.start()
            ag_descs.append(rdma)

        for d in _XOR_ORDER:
            p = me ^ d
            pltpu.make_async_remote_copy(
                src_ref=ag_buf.at[me],
                dst_ref=ag_buf.at[p],
                send_sem=ag_send.at[p],
                recv_sem=ag_recv.at[p],
                device_id=(p,),
                device_id_type=pl.DeviceIdType.MESH,
            ).wait_recv()

        for d in _XOR_ORDER:
            p = me ^ d
            pltpu.make_async_remote_copy(
                src_ref=sbuf.at[p],
                dst_ref=rs_buf.at[me],
                send_sem=rs_send.at[p],
                recv_sem=rs_recv.at[me],
                device_id=(p,),
                device_id_type=pl.DeviceIdType.MESH,
            ).wait_send()
        for rdma in ag_descs:
            rdma.wait_send()

        out_ref[:] = ag_buf[:].reshape(B, SQ, DM)

    out = pl.pallas_call(
        body,
        out_shape=jax.ShapeDtypeStruct((B, SQ, DM), jnp.bfloat16),
        in_specs=[pl.BlockSpec(memory_space=pltpu.VMEM)] * 5,
        out_specs=pl.BlockSpec(memory_space=pltpu.VMEM),
        scratch_shapes=[
            pltpu.VMEM((N_DEV, RB, DM), jnp.bfloat16),
            pltpu.VMEM((N_DEV, RB, DM), jnp.bfloat16),
            pltpu.VMEM((N_DEV, RB, DM), jnp.bfloat16),
            pltpu.SemaphoreType.DMA((N_DEV,)),
            pltpu.SemaphoreType.DMA((N_DEV,)),
            pltpu.SemaphoreType.DMA((N_DEV,)),
            pltpu.SemaphoreType.DMA((N_DEV,)),
        ],
        compiler_params=pltpu.CompilerParams(collective_id=0),
    )(xm, wq_loc, kb, vb, wo_loc)
    return out


# device time: 17374 ns/iter; 1.0700x vs baseline; 1.0700x over previous
import jax
import jax.numpy as jnp
from jax import lax
from jax.experimental import pallas as pl
from jax.experimental.pallas import tpu as pltpu

N_DEV = 8
B = 2
SQ = 128
SKV = 128
HQ = 32
DH = 64
DM = 512
H_LOC = HQ // N_DEV
CH = H_LOC * DH
RB = (B * SQ) // N_DEV
BLK_PER_B = SQ // RB

_XOR_ORDER = [6, 2, 5, 7, 1, 3, 4]


def kernel(x, Wq, K_ext, V_ext, Wo):
    my = lax.axis_index("i")
    z = jnp.int32(0)
    bf = jnp.bfloat16
    wq_loc = lax.dynamic_slice(Wq, (z, my * CH), (DM, CH)).astype(bf)
    wo_loc = lax.dynamic_slice(Wo, (my * CH, z), (CH, DM)).astype(bf)
    xm = x.reshape(B * SQ, DM).astype(bf)
    kb = K_ext.astype(bf)
    vb = V_ext.astype(bf)

    def body(x_ref, wq_ref, k_ref, v_ref, wo_ref, out_ref,
             sbuf, rs_buf, ag_buf,
             rs_send, rs_recv, ag_send, ag_recv):
        me = lax.axis_index("i")

        barrier = pltpu.get_barrier_semaphore()
        for d in range(1, N_DEV):
            pl.semaphore_signal(
                barrier, inc=1,
                device_id=(me ^ d,),
                device_id_type=pl.DeviceIdType.MESH,
            )

        q = jnp.dot(x_ref[:], wq_ref[:],
                    preferred_element_type=jnp.float32)
        q = (q * 0.125).astype(jnp.bfloat16)

        ii = lax.broadcasted_iota(jnp.int32, (SQ, SKV), 0)
        jj = lax.broadcasted_iota(jnp.int32, (SQ, SKV), 1)
        mask = (jj // 64) <= (ii // 64)

        kv = k_ref[:]
        vv = v_ref[:]
        wo = wo_ref[:]

        rows = []
        for b in range(B):
            blocks = []
            for h in range(H_LOC):
                qbh = q[b * SQ:(b + 1) * SQ, h * DH:(h + 1) * DH]
                kbh = kv[b, :, h, :]
                vbh = vv[b, :, h, :]
                scores = lax.dot_general(
                    qbh, kbh, (((1,), (1,)), ((), ())),
                    preferred_element_type=jnp.float32,
                )
                w = jnp.exp(jnp.where(mask, scores, -1e9))
                s = jnp.sum(w, axis=1, keepdims=True)
                ctx = jnp.dot(w.astype(jnp.bfloat16), vbh,
                              preferred_element_type=jnp.float32)
                blocks.append((ctx / s).astype(jnp.bfloat16))
            rows.append(jnp.concatenate(blocks, axis=1))
        ctx_all = jnp.concatenate(rows, axis=0)

        partial = jnp.dot(ctx_all, wo,
                          preferred_element_type=jnp.float32)
        sbuf[:] = partial.reshape(N_DEV, RB, DM).astype(jnp.bfloat16)

        pl.semaphore_wait(barrier, N_DEV - 1)

        for d in _XOR_ORDER:
            p = me ^ d
            pltpu.make_async_remote_copy(
                src_ref=sbuf.at[p],
                dst_ref=rs_buf.at[me],
                send_sem=rs_send.at[p],
                recv_sem=rs_recv.at[me],
                device_id=(p,),
                device_id_type=pl.DeviceIdType.MESH,
            ).start()
        rs_buf[me] = sbuf[me]

        for d in _XOR_ORDER:
            p = me ^ d
            pltpu.make_async_remote_copy(
                src_ref=sbuf.at[p],
                dst_ref=rs_buf.at[p],
                send_sem=rs_send.at[p],
                recv_sem=rs_recv.at[p],
                device_id=(p,),
                device_id_type=pl.DeviceIdType.MESH,
            ).wait_recv()

        red = jnp.sum(rs_buf[:].astype(jnp.float32), axis=0)
        ag_buf[me] = red.astype(jnp.bfloat16)

        ag_descs = []
        for d in _XOR_ORDER:
            p = me ^ d
            rdma = pltpu.make_async_remote_copy(
                src_ref=ag_buf.at[me],
                dst_ref=ag_buf.at[me],
                send_sem=ag_send.at[p],
                recv_sem=ag_recv.at[me],
                device_id=(p,),
                device_id_type=pl.DeviceIdType.MESH,
            )
            rdma.start()
            ag_descs.append(rdma)

        for d in _XOR_ORDER:
            p = me ^ d
            pltpu.make_async_remote_copy(
                src_ref=ag_buf.at[me],
                dst_ref=ag_buf.at[p],
                send_sem=ag_send.at[p],
                recv_sem=ag_recv.at[p],
                device_id=(p,),
                device_id_type=pl.DeviceIdType.MESH,
            ).wait_recv()

        for d in _XOR_ORDER:
            p = me ^ d
            pltpu.make_async_remote_copy(
                src_ref=sbuf.at[p],
                dst_ref=rs_buf.at[me],
                send_sem=rs_send.at[p],
                recv_sem=rs_recv.at[me],
                device_id=(p,),
                device_id_type=pl.DeviceIdType.MESH,
            ).wait_send()
        for rdma in ag_descs:
            rdma.wait_send()

        out_ref[:] = ag_buf[:].reshape(B, SQ, DM)

    out = pl.pallas_call(
        body,
        out_shape=jax.ShapeDtypeStruct((B, SQ, DM), jnp.bfloat16),
        in_specs=[pl.BlockSpec(memory_space=pltpu.VMEM)] * 5,
        out_specs=pl.BlockSpec(memory_space=pltpu.VMEM),
        scratch_shapes=[
            pltpu.VMEM((N_DEV, RB, DM), jnp.bfloat16),
            pltpu.VMEM((N_DEV, RB, DM), jnp.bfloat16),
            pltpu.VMEM((N_DEV, RB, DM), jnp.bfloat16),
            pltpu.SemaphoreType.DMA((N_DEV,)),
            pltpu.SemaphoreType.DMA((N_DEV,)),
            pltpu.SemaphoreType.DMA((N_DEV,)),
            pltpu.SemaphoreType.DMA((N_DEV,)),
        ],
        compiler_params=pltpu.CompilerParams(collective_id=0),
    )(xm, wq_loc, kb, vb, wo_loc)
    return out
